# baseline (speedup 1.0000x reference)
import functools

import jax
import jax.numpy as jnp
from jax import lax
from jax.experimental import pallas as pl
from jax.experimental.pallas import tpu as pltpu
from jax.experimental.pallas import tpu_sc as plsc

V = 50000
C = 128
R = 9
U = 128
RA = 5

V_PAD = 50176
N_WORKERS = 32
V_TILE = V_PAD // N_WORKERS
VCH = 32
NCHK = V_TILE // VCH
NU = U // 16


BVY = 2000
NBY = V // BVY


def _make_matmul(nr):
    def body(n_ref, w_ref, *o_refs):
        part = jnp.dot(n_ref[...], w_ref[...],
                       preferred_element_type=jnp.float32)
        for r in range(nr):
            o_refs[r][...] = part[:, r * U:(r + 1) * U]

    return pl.pallas_call(
        body,
        grid=(NBY,),
        in_specs=[
            pl.BlockSpec((BVY, C), lambda i: (i, 0)),
            pl.BlockSpec((C, nr * U), lambda i: (0, 0)),
        ],
        out_specs=[pl.BlockSpec((BVY, U), lambda i: (i, 0))
                   for _ in range(nr)],
        out_shape=[jax.ShapeDtypeStruct((V, U), jnp.float32)
                   for _ in range(nr)],
        compiler_params=pltpu.CompilerParams(
            dimension_semantics=("arbitrary",),
        ),
    )


_matmul_a = _make_matmul(RA)
_matmul_b = _make_matmul(R - RA)


def _make_combine(nr, r0, final):
    nsrc = nr + (1 if final else 0)

    def body(idx_hbm, bias_hbm, *rest):
        y_hbms = rest[:nr]
        acc_hbm = rest[nr] if final else None
        out_hbm = rest[nr + (1 if final else 0)]
        idx_v, bias_v, rows_v, out_v, g0, g1, s0, s1 = \
            rest[nr + (2 if final else 1):]
        gsems = (g0, g1)
        ssems = (s0, s1)
        wid = lax.axis_index("s") * 2 + lax.axis_index("c")
        vb0 = wid * V_TILE

        for r in range(nr):
            pltpu.sync_copy(
                idx_hbm.at[pl.ds((r0 + r) * V_PAD + vb0, V_TILE)],
                idx_v.at[pl.ds(r * V_TILE, V_TILE)])
        pltpu.sync_copy(bias_hbm, bias_v)
        bias_regs = [bias_v[pl.ds(u * 16, 16)] for u in range(NU)]

        def descs(k, b):
            d = [pltpu.make_async_copy(
                     y_hbms[r].at[idx_v.at[pl.ds(r * V_TILE + k * VCH, VCH)]],
                     rows_v.at[b, r], gsems[b]) for r in range(nr)]
            if final:
                d.append(pltpu.make_async_copy(
                    acc_hbm.at[pl.ds(vb0 + k * VCH, VCH)],
                    rows_v.at[b, nr], gsems[b]))
            return d

        def start_gathers(k, b):
            for d in descs(k, b):
                d.start()

        def wait_gathers(k, b):
            for d in descs(k, b):
                d.wait()

        def store_desc(k, o):
            return pltpu.make_async_copy(
                out_v.at[o], out_hbm.at[pl.ds(vb0 + k * VCH, VCH)], ssems[o])

        def accumulate(b, o):
            def row(j, carry):
                for u in range(NU):
                    acc = rows_v[b, 0, j, pl.ds(u * 16, 16)]
                    for r in range(1, nsrc):
                        acc = acc + rows_v[b, r, j, pl.ds(u * 16, 16)]
                    if final:
                        acc = jnp.maximum(acc + bias_regs[u], 0.0)
                    out_v[o, j, pl.ds(u * 16, 16)] = acc
                return carry
            lax.fori_loop(0, VCH, row, 0)

        def chunk(k, b):
            wait_gathers(k, b)

            def _next(k=k, b=b):
                start_gathers(k + 1, 1 - b)
            pl.when(k + 1 < NCHK)(_next)

            o = b

            def _wait_store(k=k, o=o):
                store_desc(k - 2, o).wait()
            pl.when(k >= 2)(_wait_store)
            accumulate(b, o)
            store_desc(k, o).start()

        start_gathers(0, 0)

        def pair(t, carry):
            chunk(2 * t, 0)
            chunk(2 * t + 1, 1)
            return carry

        lax.fori_loop(0, NCHK // 2, pair, 0)
        chunk(NCHK - 1, 0)
        store_desc(NCHK - 2, 1).wait()
        store_desc(NCHK - 1, 0).wait()

    return functools.partial(
        pl.kernel,
        mesh=plsc.VectorSubcoreMesh(core_axis_name="c", subcore_axis_name="s"),
        out_type=jax.ShapeDtypeStruct((V_PAD, U), jnp.float32),
        scratch_types=[
            pltpu.VMEM((nr * V_TILE,), jnp.int32),
            pltpu.VMEM((U,), jnp.float32),
            pltpu.VMEM((2, nsrc, VCH, U), jnp.float32),
            pltpu.VMEM((2, VCH, U), jnp.float32),
            pltpu.SemaphoreType.DMA,
            pltpu.SemaphoreType.DMA,
            pltpu.SemaphoreType.DMA,
            pltpu.SemaphoreType.DMA,
        ],
    )(body)


_combine_a = _make_combine(RA, 0, final=False)
_combine_b = _make_combine(R - RA, RA, final=True)


def kernel(nodes, nodes_indices, column_indices, weights, bias):
    m, v, c = nodes.shape
    nodes_bf = nodes.reshape(v, c).astype(jnp.bfloat16)
    w2 = (weights.reshape(R, C, U).transpose(1, 0, 2)
          .reshape(C, R * U).astype(jnp.bfloat16))
    src = nodes_indices[:, 1].reshape(v, R).T
    idx = jnp.pad(src, ((0, 0), (0, V_PAD - v))).reshape(-1)
    ys_a = _matmul_a(nodes_bf, w2[:, :RA * U])
    acc = _combine_a(idx, bias, *ys_a)
    ys_b = _matmul_b(nodes_bf, w2[:, RA * U:])
    out = _combine_b(idx, bias, acc, *ys_b)
    return out[:v].reshape(m, v, U)


# speedup vs baseline: 1.1537x; 1.1537x over previous
import functools

import jax
import jax.numpy as jnp
from jax import lax
from jax.experimental import pallas as pl
from jax.experimental.pallas import tpu as pltpu
from jax.experimental.pallas import tpu_sc as plsc

V = 50000
C = 128
R = 9
U = 128

V_PAD = 50176
N_WORKERS = 32
V_TILE = V_PAD // N_WORKERS
VCH = 32
NCHK = V_TILE // VCH
NU = U // 16


BVY = 1000
NBY = V // BVY


def _mm_body(n_ref, w_ref, *o_refs):
    part = jnp.dot(n_ref[...], w_ref[...],
                   preferred_element_type=jnp.float32)
    for r in range(R):
        o_refs[r][...] = part[:, r * U:(r + 1) * U]


_matmul = pl.pallas_call(
    _mm_body,
    grid=(NBY,),
    in_specs=[
        pl.BlockSpec((BVY, C), lambda i: (i, 0)),
        pl.BlockSpec((C, R * U), lambda i: (0, 0)),
    ],
    out_specs=[pl.BlockSpec((BVY, U), lambda i: (i, 0)) for _ in range(R)],
    out_shape=[jax.ShapeDtypeStruct((V, U), jnp.float32) for _ in range(R)],
    compiler_params=pltpu.CompilerParams(
        dimension_semantics=("arbitrary",),
    ),
)


def _combine_body(idx_hbm, bias_hbm, *rest):
    y_hbms = rest[:R]
    out_hbm = rest[R]
    idx_v, bias_v, rows_v, out_v, gsem0, gsem1, ssem0, ssem1 = rest[R + 1:]
    gsems = (gsem0, gsem1)
    ssems = (ssem0, ssem1)
    wid = lax.axis_index("s") * 2 + lax.axis_index("c")
    vb0 = wid * V_TILE

    for r in range(R):
        pltpu.sync_copy(idx_hbm.at[pl.ds(r * V_PAD + vb0, V_TILE)],
                        idx_v.at[pl.ds(r * V_TILE, V_TILE)])
    pltpu.sync_copy(bias_hbm, bias_v)
    bias_regs = [bias_v[pl.ds(u * 16, 16)] for u in range(NU)]

    def gather_desc(k, r, b):
        return pltpu.make_async_copy(
            y_hbms[r].at[idx_v.at[pl.ds(r * V_TILE + k * VCH, VCH)]],
            rows_v.at[b, r], gsems[b])

    def start_gathers(k, b):
        for r in range(R):
            gather_desc(k, r, b).start()

    def wait_gathers(k, b):
        for r in range(R):
            gather_desc(k, r, b).wait()

    def store_desc(k, o):
        return pltpu.make_async_copy(
            out_v.at[o], out_hbm.at[pl.ds(vb0 + k * VCH, VCH)], ssems[o])

    def accumulate(b, o):
        def row(j, carry):
            for u in range(NU):
                acc = rows_v[b, 0, j, pl.ds(u * 16, 16)]
                for r in range(1, R):
                    acc = acc + rows_v[b, r, j, pl.ds(u * 16, 16)]
                out_v[o, j, pl.ds(u * 16, 16)] = jnp.maximum(
                    acc + bias_regs[u], 0.0)
            return carry
        lax.fori_loop(0, VCH, row, 0)

    def chunk(k, b):
        wait_gathers(k, b)

        def _next(k=k, b=b):
            start_gathers(k + 1, 1 - b)
        pl.when(k + 1 < NCHK)(_next)

        o = b

        def _wait_store(k=k, o=o):
            store_desc(k - 2, o).wait()
        pl.when(k >= 2)(_wait_store)
        accumulate(b, o)
        store_desc(k, o).start()

    start_gathers(0, 0)

    def pair(t, carry):
        chunk(2 * t, 0)
        chunk(2 * t + 1, 1)
        return carry

    lax.fori_loop(0, NCHK // 2, pair, 0)
    chunk(NCHK - 1, 0)
    store_desc(NCHK - 2, 1).wait()
    store_desc(NCHK - 1, 0).wait()


_combine = functools.partial(
    pl.kernel,
    mesh=plsc.VectorSubcoreMesh(core_axis_name="c", subcore_axis_name="s"),
    out_type=jax.ShapeDtypeStruct((V_PAD, U), jnp.float32),
    scratch_types=[
        pltpu.VMEM((R * V_TILE,), jnp.int32),
        pltpu.VMEM((U,), jnp.float32),
        pltpu.VMEM((2, R, VCH, U), jnp.float32),
        pltpu.VMEM((2, VCH, U), jnp.float32),
        pltpu.SemaphoreType.DMA,
        pltpu.SemaphoreType.DMA,
        pltpu.SemaphoreType.DMA,
        pltpu.SemaphoreType.DMA,
    ],
)(_combine_body)


def kernel(nodes, nodes_indices, column_indices, weights, bias):
    m, v, c = nodes.shape
    nodes_bf = nodes.reshape(v, c).astype(jnp.bfloat16)
    w2 = (weights.reshape(R, C, U).transpose(1, 0, 2)
          .reshape(C, R * U).astype(jnp.bfloat16))
    ys = _matmul(nodes_bf, w2)
    src = nodes_indices[:, 1].reshape(v, R).T
    idx = jnp.pad(src, ((0, 0), (0, V_PAD - v))).reshape(-1)
    out = _combine(idx, bias, *ys)
    return out[:v].reshape(m, v, U)


# speedup vs baseline: 1.1851x; 1.0272x over previous
import functools

import jax
import jax.numpy as jnp
from jax import lax
from jax.experimental import pallas as pl
from jax.experimental.pallas import tpu as pltpu
from jax.experimental.pallas import tpu_sc as plsc

V = 50000
C = 128
R = 9
U = 128

V_PAD = 50176
N_WORKERS = 32
V_TILE = V_PAD // N_WORKERS
VCH = 32
NCHK = V_TILE // VCH
NU = U // 16


BVY = 2000
NBY = V // BVY


def _mm_body(n_ref, w_ref, *o_refs):
    part = jnp.dot(n_ref[...].astype(jnp.bfloat16), w_ref[...],
                   preferred_element_type=jnp.float32)
    for r in range(R):
        o_refs[r][...] = part[:, r * U:(r + 1) * U]


_matmul = pl.pallas_call(
    _mm_body,
    grid=(NBY,),
    in_specs=[
        pl.BlockSpec((BVY, C), lambda i: (i, 0)),
        pl.BlockSpec((C, R * U), lambda i: (0, 0)),
    ],
    out_specs=[pl.BlockSpec((BVY, U), lambda i: (i, 0)) for _ in range(R)],
    out_shape=[jax.ShapeDtypeStruct((V, U), jnp.float32) for _ in range(R)],
    compiler_params=pltpu.CompilerParams(
        dimension_semantics=("arbitrary",),
    ),
)


def _combine_body(idx_hbm, bias_hbm, *rest):
    y_hbms = rest[:R]
    out_hbm = rest[R]
    idx_v, bias_v, rows_v, out_v, gsem0, gsem1, ssem0, ssem1 = rest[R + 1:]
    gsems = (gsem0, gsem1)
    ssems = (ssem0, ssem1)
    wid = lax.axis_index("s") * 2 + lax.axis_index("c")
    vb0 = wid * V_TILE

    for r in range(R):
        pltpu.sync_copy(idx_hbm.at[pl.ds(r * V_PAD + vb0, V_TILE)],
                        idx_v.at[pl.ds(r * V_TILE, V_TILE)])
    pltpu.sync_copy(bias_hbm, bias_v)
    bias_regs = [bias_v[pl.ds(u * 16, 16)] for u in range(NU)]

    def gather_desc(k, r, b):
        return pltpu.make_async_copy(
            y_hbms[r].at[idx_v.at[pl.ds(r * V_TILE + k * VCH, VCH)]],
            rows_v.at[b, r], gsems[b])

    def start_gathers(k, b):
        for r in range(R):
            gather_desc(k, r, b).start()

    def wait_gathers(k, b):
        for r in range(R):
            gather_desc(k, r, b).wait()

    def store_desc(k, o):
        return pltpu.make_async_copy(
            out_v.at[o], out_hbm.at[pl.ds(vb0 + k * VCH, VCH)], ssems[o])

    def accumulate(b, o):
        def row(j, carry):
            for u in range(NU):
                acc = rows_v[b, 0, j, pl.ds(u * 16, 16)]
                for r in range(1, R):
                    acc = acc + rows_v[b, r, j, pl.ds(u * 16, 16)]
                out_v[o, j, pl.ds(u * 16, 16)] = jnp.maximum(
                    acc + bias_regs[u], 0.0)
            return carry
        lax.fori_loop(0, VCH, row, 0)

    def chunk(k, b):
        wait_gathers(k, b)

        def _next(k=k, b=b):
            start_gathers(k + 1, 1 - b)
        pl.when(k + 1 < NCHK)(_next)

        o = b

        def _wait_store(k=k, o=o):
            store_desc(k - 2, o).wait()
        pl.when(k >= 2)(_wait_store)
        accumulate(b, o)
        store_desc(k, o).start()

    start_gathers(0, 0)

    def pair(t, carry):
        chunk(2 * t, 0)
        chunk(2 * t + 1, 1)
        return carry

    lax.fori_loop(0, NCHK // 2, pair, 0)
    chunk(NCHK - 1, 0)
    store_desc(NCHK - 2, 1).wait()
    store_desc(NCHK - 1, 0).wait()


_combine = functools.partial(
    pl.kernel,
    mesh=plsc.VectorSubcoreMesh(core_axis_name="c", subcore_axis_name="s"),
    out_type=jax.ShapeDtypeStruct((V_PAD, U), jnp.float32),
    scratch_types=[
        pltpu.VMEM((R * V_TILE,), jnp.int32),
        pltpu.VMEM((U,), jnp.float32),
        pltpu.VMEM((2, R, VCH, U), jnp.float32),
        pltpu.VMEM((2, VCH, U), jnp.float32),
        pltpu.SemaphoreType.DMA,
        pltpu.SemaphoreType.DMA,
        pltpu.SemaphoreType.DMA,
        pltpu.SemaphoreType.DMA,
    ],
)(_combine_body)


def kernel(nodes, nodes_indices, column_indices, weights, bias):
    m, v, c = nodes.shape
    nodes2 = nodes.reshape(v, c)
    w2 = (weights.reshape(R, C, U).transpose(1, 0, 2)
          .reshape(C, R * U).astype(jnp.bfloat16))
    ys = _matmul(nodes2, w2)
    src = nodes_indices[:, 1].reshape(v, R).T
    idx = jnp.pad(src, ((0, 0), (0, V_PAD - v))).reshape(-1)
    out = _combine(idx, bias, *ys)
    return out[:v].reshape(m, v, U)


# speedup vs baseline: 1.2470x; 1.0523x over previous
import functools

import jax
import jax.numpy as jnp
from jax import lax
from jax.experimental import pallas as pl
from jax.experimental.pallas import tpu as pltpu
from jax.experimental.pallas import tpu_sc as plsc

V = 50000
C = 128
R = 9
U = 128

V_PAD = 50176
N_WORKERS = 32
V_TILE = V_PAD // N_WORKERS
VCH = 32
NCHK = V_TILE // VCH
NU = U // 16


BVY = 2000
NBY = V // BVY


def _mm_body(n_ref, w_ref, *o_refs):
    part = jnp.dot(n_ref[...].astype(jnp.bfloat16), w_ref[...],
                   preferred_element_type=jnp.float32)
    for r in range(R):
        o_refs[r][...] = part[:, r * U:(r + 1) * U]


_matmul = pl.pallas_call(
    _mm_body,
    grid=(NBY,),
    in_specs=[
        pl.BlockSpec((BVY, C), lambda i: (i, 0)),
        pl.BlockSpec((C, R * U), lambda i: (0, 0)),
    ],
    out_specs=[pl.BlockSpec((BVY, U), lambda i: (i, 0)) for _ in range(R)],
    out_shape=[jax.ShapeDtypeStruct((V, U), jnp.float32) for _ in range(R)],
    compiler_params=pltpu.CompilerParams(
        dimension_semantics=("arbitrary",),
    ),
)


def _combine_body(idx_hbm, bias_hbm, *rest):
    y_hbms = rest[:R]
    out_hbm = rest[R]
    idx_v, bias_v, rows_v, out_v, gsem0, gsem1, ssem0, ssem1 = rest[R + 1:]
    gsems = (gsem0, gsem1)
    ssems = (ssem0, ssem1)
    wid = lax.axis_index("s") * 2 + lax.axis_index("c")
    vb0 = wid * V_TILE

    for r in range(R):
        pltpu.sync_copy(idx_hbm.at[pl.ds(r * V_PAD + vb0, V_TILE)],
                        idx_v.at[pl.ds(r * V_TILE, V_TILE)])
    pltpu.sync_copy(bias_hbm, bias_v)
    bias_regs = [bias_v[pl.ds(u * 16, 16)] for u in range(NU)]

    def gather_desc(k, r, b):
        return pltpu.make_async_copy(
            y_hbms[r].at[idx_v.at[pl.ds(r * V_TILE + k * VCH, VCH)]],
            rows_v.at[b, r], gsems[b])

    def start_gathers(k, b):
        for r in range(R):
            gather_desc(k, r, b).start()

    def wait_gathers(k, b):
        for r in range(R):
            gather_desc(k, r, b).wait()

    def store_desc(k, o):
        return pltpu.make_async_copy(
            out_v.at[o], out_hbm.at[pl.ds(vb0 + k * VCH, VCH)], ssems[o])

    def accumulate(b, o):
        def row(j, carry):
            for u in range(NU):
                acc = rows_v[b, 0, j, pl.ds(u * 16, 16)]
                for r in range(1, R):
                    acc = acc + rows_v[b, r, j, pl.ds(u * 16, 16)]
                out_v[o, j, pl.ds(u * 16, 16)] = jnp.maximum(
                    acc + bias_regs[u], 0.0)
            return carry
        lax.fori_loop(0, VCH, row, 0)

    def chunk(k, b):
        def _next(k=k, b=b):
            start_gathers(k + 1, 1 - b)
        pl.when(k + 1 < NCHK)(_next)

        wait_gathers(k, b)
        o = b

        def _wait_store(k=k, o=o):
            store_desc(k - 2, o).wait()
        pl.when(k >= 2)(_wait_store)
        accumulate(b, o)
        store_desc(k, o).start()

    start_gathers(0, 0)

    def pair(t, carry):
        chunk(2 * t, 0)
        chunk(2 * t + 1, 1)
        return carry

    lax.fori_loop(0, NCHK // 2, pair, 0)
    chunk(NCHK - 1, 0)
    store_desc(NCHK - 2, 1).wait()
    store_desc(NCHK - 1, 0).wait()


_combine = functools.partial(
    pl.kernel,
    mesh=plsc.VectorSubcoreMesh(core_axis_name="c", subcore_axis_name="s"),
    out_type=jax.ShapeDtypeStruct((V_PAD, U), jnp.float32),
    scratch_types=[
        pltpu.VMEM((R * V_TILE,), jnp.int32),
        pltpu.VMEM((U,), jnp.float32),
        pltpu.VMEM((2, R, VCH, U), jnp.float32),
        pltpu.VMEM((2, VCH, U), jnp.float32),
        pltpu.SemaphoreType.DMA,
        pltpu.SemaphoreType.DMA,
        pltpu.SemaphoreType.DMA,
        pltpu.SemaphoreType.DMA,
    ],
)(_combine_body)


def kernel(nodes, nodes_indices, column_indices, weights, bias):
    m, v, c = nodes.shape
    nodes2 = nodes.reshape(v, c)
    w2 = (weights.reshape(R, C, U).transpose(1, 0, 2)
          .reshape(C, R * U).astype(jnp.bfloat16))
    ys = _matmul(nodes2, w2)
    src = nodes_indices[:, 1].reshape(v, R).T
    idx = jnp.pad(src, ((0, 0), (0, V_PAD - v))).reshape(-1)
    out = _combine(idx, bias, *ys)
    return out[:v].reshape(m, v, U)
